# Initial kernel scaffold; baseline (speedup 1.0000x reference)
import jax
import jax.numpy as jnp
from jax import lax
from jax.experimental import pallas as pl
from jax.experimental.pallas import tpu as pltpu

_B, _S, _D = 4, 8192, 4096
_E, _K = 64, 8
_BT = 512


def _gate_kernel(x_ref, w_ref, tw_ref, ti_ref):
    x = x_ref[...]
    w = w_ref[...]
    logits = lax.dot_general(
        x, w, (((1,), (1,)), ((), ())),
        preferred_element_type=jnp.float32)

    bt = logits.shape[0]
    v1 = logits[:, :, None]
    v2 = logits[:, None, :]
    e1 = lax.broadcasted_iota(jnp.int32, (bt, _E, _E), 1)
    e2 = lax.broadcasted_iota(jnp.int32, (bt, _E, _E), 2)
    ahead = (v2 > v1) | ((v2 == v1) & (e2 < e1))
    rank = jnp.sum(ahead.astype(jnp.int32), axis=2)

    krow = lax.broadcasted_iota(jnp.int32, (bt, _K, _E), 1)
    sel = (rank[:, None, :] == krow)
    eidx = lax.broadcasted_iota(jnp.int32, (bt, _K, _E), 2)
    idx_k = jnp.sum(jnp.where(sel, eidx, 0), axis=2)
    logit_k = jnp.sum(jnp.where(sel, logits[:, None, :], 0.0), axis=2)

    wts = jax.nn.sigmoid(logit_k)
    wts = wts / jnp.sum(wts, axis=-1, keepdims=True)
    tw_ref[...] = wts
    ti_ref[...] = idx_k


def kernel(x, W):
    T = _B * _S
    xf = x.reshape(T, _D)
    grid = (T // _BT,)
    tw, ti = pl.pallas_call(
        _gate_kernel,
        grid=grid,
        in_specs=[
            pl.BlockSpec((_BT, _D), lambda i: (i, 0)),
            pl.BlockSpec((_E, _D), lambda i: (0, 0)),
        ],
        out_specs=[
            pl.BlockSpec((_BT, _K), lambda i: (i, 0)),
            pl.BlockSpec((_BT, _K), lambda i: (i, 0)),
        ],
        out_shape=[
            jax.ShapeDtypeStruct((T, _K), jnp.float32),
            jax.ShapeDtypeStruct((T, _K), jnp.int32),
        ],
    )(xf, W)
    return tw.reshape(_B, _S, _K), ti.reshape(_B, _S, _K)


# speedup vs baseline: 1.4564x; 1.4564x over previous
import jax
import jax.numpy as jnp
from jax import lax
from jax.experimental import pallas as pl
from jax.experimental.pallas import tpu as pltpu

_B, _S, _D = 4, 8192, 4096
_E, _K = 64, 8
_BT = 512


def _gate_kernel(x_ref, w_ref, tw_ref, ti_ref):
    x = x_ref[...]
    w = w_ref[...]
    logits = lax.dot_general(
        x, w, (((1,), (1,)), ((), ())),
        preferred_element_type=jnp.float32)

    bt = logits.shape[0]
    eiota = lax.broadcasted_iota(jnp.int32, (bt, _E), 1)
    kcol = lax.broadcasted_iota(jnp.int32, (bt, _K), 1)
    idx_k = jnp.zeros((bt, _K), jnp.int32)
    logit_k = jnp.zeros((bt, _K), jnp.float32)
    work = logits
    for k in range(_K):
        m = jnp.max(work, axis=-1, keepdims=True)
        amax = jnp.min(jnp.where(work == m, eiota, _E),
                       axis=-1, keepdims=True)
        idx_k = jnp.where(kcol == k, amax, idx_k)
        logit_k = jnp.where(kcol == k, m, logit_k)
        work = jnp.where(eiota == amax, -jnp.inf, work)

    wts = jax.nn.sigmoid(logit_k)
    wts = wts / jnp.sum(wts, axis=-1, keepdims=True)
    tw_ref[...] = wts
    ti_ref[...] = idx_k


def kernel(x, W):
    T = _B * _S
    xf = x.reshape(T, _D)
    grid = (T // _BT,)
    tw, ti = pl.pallas_call(
        _gate_kernel,
        grid=grid,
        in_specs=[
            pl.BlockSpec((_BT, _D), lambda i: (i, 0)),
            pl.BlockSpec((_E, _D), lambda i: (0, 0)),
        ],
        out_specs=[
            pl.BlockSpec((_BT, _K), lambda i: (i, 0)),
            pl.BlockSpec((_BT, _K), lambda i: (i, 0)),
        ],
        out_shape=[
            jax.ShapeDtypeStruct((T, _K), jnp.float32),
            jax.ShapeDtypeStruct((T, _K), jnp.int32),
        ],
    )(xf, W)
    return tw.reshape(_B, _S, _K), ti.reshape(_B, _S, _K)


# speedup vs baseline: 1.7768x; 1.2200x over previous
import jax
import jax.numpy as jnp
from jax import lax
from jax.experimental import pallas as pl
from jax.experimental.pallas import tpu as pltpu

_B, _S, _D = 4, 8192, 4096
_E, _K = 64, 8
_BT = 512


def _gate_kernel(x_ref, w_ref, tw_ref, ti_ref):
    x = x_ref[...]
    w = w_ref[...]
    logits = lax.dot_general(
        x, w, (((1,), (1,)), ((), ())),
        preferred_element_type=jnp.float32)

    bt = logits.shape[0]
    g = jax.nn.sigmoid(logits)
    gb = lax.bitcast_convert_type(g, jnp.int32)
    eiota = lax.broadcasted_iota(jnp.int32, (bt, _E), 1)
    packed = jnp.bitwise_or(jnp.bitwise_and(gb, jnp.int32(-64)),
                            jnp.int32(63) - eiota)
    workf = lax.bitcast_convert_type(packed, jnp.float32)

    kcol = lax.broadcasted_iota(jnp.int32, (bt, _K), 1)
    sel = jnp.zeros((bt, _K), jnp.float32)
    for k in range(_K):
        m = jnp.max(workf, axis=-1, keepdims=True)
        sel = jnp.where(kcol == k, m, sel)
        workf = jnp.where(workf == m, -1.0, workf)

    selb = lax.bitcast_convert_type(sel, jnp.int32)
    idx_k = jnp.int32(63) - jnp.bitwise_and(selb, jnp.int32(63))
    wts = lax.bitcast_convert_type(
        jnp.bitwise_or(jnp.bitwise_and(selb, jnp.int32(-64)), jnp.int32(32)),
        jnp.float32)
    wts = wts / jnp.sum(wts, axis=-1, keepdims=True)
    tw_ref[...] = wts
    ti_ref[...] = idx_k


def kernel(x, W):
    T = _B * _S
    xf = x.reshape(T, _D)
    grid = (T // _BT,)
    tw, ti = pl.pallas_call(
        _gate_kernel,
        grid=grid,
        in_specs=[
            pl.BlockSpec((_BT, _D), lambda i: (i, 0)),
            pl.BlockSpec((_E, _D), lambda i: (0, 0)),
        ],
        out_specs=[
            pl.BlockSpec((_BT, _K), lambda i: (i, 0)),
            pl.BlockSpec((_BT, _K), lambda i: (i, 0)),
        ],
        out_shape=[
            jax.ShapeDtypeStruct((T, _K), jnp.float32),
            jax.ShapeDtypeStruct((T, _K), jnp.int32),
        ],
    )(xf, W)
    return tw.reshape(_B, _S, _K), ti.reshape(_B, _S, _K)
